# baseline (speedup 1.0000x reference)
import jax
import jax.numpy as jnp
from jax import lax
from jax.experimental import pallas as pl
from jax.experimental.pallas import tpu as pltpu
from jax.experimental.pallas import tpu_sc as plsc

M = 100000
D = 256
B = 16384
C = 100
CP = 128

NW = 32
RNG = 3128
CAP = 1024
PIECE = 128
NPIECE = CAP // PIECE
SRCN = 3136
CS = 136
NCH = RNG // CS


def _padx_body(xlg_r, oxlg_r):
    oxlg_r[:, :C] = xlg_r[...]


def _tc_padx(xlg):
    grid = 128
    bb = B // grid
    return pl.pallas_call(
        _padx_body,
        grid=(grid,),
        in_specs=[pl.BlockSpec((bb, C), lambda i: (i, 0))],
        out_specs=pl.BlockSpec((bb, CP), lambda i: (i, 0)),
        out_shape=jax.ShapeDtypeStruct((B, CP), jnp.float32),
    )(xlg)


def _tc_padbuf(lg):
    grid = 100
    bm = M // grid
    return pl.pallas_call(
        _padx_body,
        grid=(grid,),
        in_specs=[pl.BlockSpec((bm, C), lambda i: (i, 0))],
        out_specs=pl.BlockSpec((bm, CP), lambda i: (i, 0)),
        out_shape=jax.ShapeDtypeStruct((M, CP), jnp.float32),
    )(lg)


def _unpad_body(p_r, o_r):
    o_r[...] = p_r[:, :C]


def _tc_unpad(p):
    grid = 100
    bm = M // grid
    return pl.pallas_call(
        _unpad_body,
        grid=(grid,),
        in_specs=[pl.BlockSpec((bm, CP), lambda i: (i, 0))],
        out_specs=pl.BlockSpec((bm, C), lambda i: (i, 0)),
        out_shape=jax.ShapeDtypeStruct((M, C), jnp.float32),
    )(p)


def _sc_body(bx_h, by_h, en_h, lgp_h, x_h, y_h, eu_h, xlgp_h, idx_h,
             obx, oby, oen, olgp,
             idx_v, src_v, rowl, jl, yv, ev, cbx, clg, cby, cen):
    w = lax.axis_index("s") * 2 + lax.axis_index("c")
    base = jnp.where(w == NW - 1, M - RNG, w * RNG).astype(jnp.int32)
    lane = lax.iota(jnp.int32, 16)

    pltpu.sync_copy(idx_h, idx_v)

    def init_body(g, _):
        src_v[pl.ds(g * 16, 16)] = jnp.full((16,), -1, jnp.int32)
        return 0
    lax.fori_loop(0, SRCN // 16, init_body, 0)

    def scan_body(g, _):
        iv = idx_v[pl.ds(g * 16, 16)]
        loc = iv - base
        inr = (loc >= 0) & (loc < RNG)
        j = g * 16 + lane
        _, lastm = plsc.scan_count(loc, mask=inr)
        plsc.store_scatter(src_v, [loc], j, mask=lastm & inr)
        return 0
    lax.fori_loop(0, B // 16, scan_body, 0)

    def comp_body(g, carry):
        k, pmax = carry
        s = src_v[pl.ds(g * 16, 16)]
        m = s >= 0
        ones = jnp.where(m, 1, 0).astype(jnp.int32)
        pos = k + plsc.cumsum(ones) - 1
        m = m & (pos < CAP)
        rows = base + g * 16 + lane
        plsc.store_scatter(rowl, [pos >> 7, pos & 127], rows, mask=m)
        plsc.store_scatter(jl, [pos >> 7, pos & 127], s, mask=m)
        pair = jnp.where(m, rows * 16384 + s, -1)
        return k + jnp.sum(ones), jnp.maximum(pmax, jnp.max(pair))
    k, pmax = lax.fori_loop(0, SRCN // 16, comp_body,
                            (jnp.int32(0), jnp.int32(-1)))

    def cp_body(c, _):
        r0 = base + c * CS
        pltpu.sync_copy(bx_h.at[pl.ds(r0, CS)], cbx)
        pltpu.sync_copy(cbx, obx.at[pl.ds(r0, CS)])
        pltpu.sync_copy(lgp_h.at[pl.ds(r0, CS)], clg)
        pltpu.sync_copy(clg, olgp.at[pl.ds(r0, CS)])
        return 0
    lax.fori_loop(0, NCH, cp_body, 0)
    pltpu.sync_copy(by_h.at[pl.ds(base, RNG)], cby)
    pltpu.sync_copy(cby, oby.at[pl.ds(base, RNG)])
    pltpu.sync_copy(en_h.at[pl.ds(base, RNG)], cen)
    pltpu.sync_copy(cen, oen.at[pl.ds(base, RNG)])

    @pl.when(k > 0)
    def _():
        pad_row = pmax >> 14
        pad_j = pmax & 16383

        def pad_body(g, _):
            posv = g * 16 + lane
            m = posv >= k
            plsc.store_scatter(rowl, [posv >> 7, posv & 127],
                               jnp.full((16,), 1, jnp.int32) * pad_row, mask=m)
            plsc.store_scatter(jl, [posv >> 7, posv & 127],
                               jnp.full((16,), 1, jnp.int32) * pad_j, mask=m)
            return 0
        lax.fori_loop(k >> 4, CAP // 16, pad_body, 0)

        xb = cbx.at[pl.ds(0, PIECE)]
        lb = clg.at[pl.ds(0, PIECE)]

        def upd_body(p, _):
            pltpu.sync_copy(x_h.at[jl.at[p]], xb)
            pltpu.sync_copy(xb, obx.at[rowl.at[p]])
            pltpu.sync_copy(xlgp_h.at[jl.at[p]], lb)
            pltpu.sync_copy(lb, olgp.at[rowl.at[p]])
            pltpu.sync_copy(y_h.at[jl.at[p]], yv)
            pltpu.sync_copy(yv, oby.at[rowl.at[p]])
            pltpu.sync_copy(eu_h.at[jl.at[p]], ev)
            pltpu.sync_copy(ev, oen.at[rowl.at[p]])
            return 0
        lax.fori_loop(0, (k + PIECE - 1) >> 7, upd_body, 0)


_sc_all = pl.kernel(
    _sc_body,
    out_type=(jax.ShapeDtypeStruct((M, D), jnp.float32),
              jax.ShapeDtypeStruct((M,), jnp.int32),
              jax.ShapeDtypeStruct((M,), jnp.float32),
              jax.ShapeDtypeStruct((M, CP), jnp.float32)),
    mesh=plsc.VectorSubcoreMesh(core_axis_name="c", subcore_axis_name="s"),
    compiler_params=pltpu.CompilerParams(needs_layout_passes=False,
                                         use_tc_tiling_on_sc=False),
    scratch_types=[
        pltpu.VMEM((B,), jnp.int32),
        pltpu.VMEM((SRCN,), jnp.int32),
        pltpu.VMEM((NPIECE, PIECE), jnp.int32),
        pltpu.VMEM((NPIECE, PIECE), jnp.int32),
        pltpu.VMEM((PIECE,), jnp.int32),
        pltpu.VMEM((PIECE,), jnp.float32),
        pltpu.VMEM((CS, D), jnp.float32),
        pltpu.VMEM((CS, CP), jnp.float32),
        pltpu.VMEM((RNG,), jnp.int32),
        pltpu.VMEM((RNG,), jnp.float32),
    ],
)


@jax.jit
def kernel(bx, by_buf, ents_buf, logits_buf, x, y, ents, logits, idx):
    xlgp = _tc_padx(logits)
    lgp = _tc_padbuf(logits_buf)
    obx, oby, oen, olgp = _sc_all(bx, by_buf, ents_buf, lgp,
                                  x, y, ents, xlgp, idx)
    return obx, oby, oen, _tc_unpad(olgp)


# speedup vs baseline: 1.0021x; 1.0021x over previous
import jax
import jax.numpy as jnp
from jax import lax
from jax.experimental import pallas as pl
from jax.experimental.pallas import tpu as pltpu
from jax.experimental.pallas import tpu_sc as plsc

M = 100000
D = 256
B = 16384
C = 100
CP = 128

NW = 32
RNG = 3128
CAP = 1024
PIECE = 64
NPIECE = CAP // PIECE
SRCN = 3136
CSL = 136
NCHL = RNG // CSL
CSX = 68
NCHX = RNG // CSX


def _padx_body(xlg_r, oxlg_r):
    oxlg_r[:, :C] = xlg_r[...]


def _tc_padx(xlg):
    grid = 128
    bb = B // grid
    return pl.pallas_call(
        _padx_body,
        grid=(grid,),
        in_specs=[pl.BlockSpec((bb, C), lambda i: (i, 0))],
        out_specs=pl.BlockSpec((bb, CP), lambda i: (i, 0)),
        out_shape=jax.ShapeDtypeStruct((B, CP), jnp.float32),
    )(xlg)


def _sc_body(bx_h, by_h, en_h, lgf_h, x_h, y_h, eu_h, xlgp_h, idx_h,
             obx, oby, oen, olgf,
             idx_v, src_v, rowl, jl, yv, ev,
             cbx0, cbx1, clg0, clg1, xpiece, lgpiece, cby, cen,
             sem0, sem1):
    w = lax.axis_index("s") * 2 + lax.axis_index("c")
    base = jnp.where(w == NW - 1, M - RNG, w * RNG).astype(jnp.int32)
    lane = lax.iota(jnp.int32, 16)

    pltpu.sync_copy(idx_h, idx_v)

    def init_body(g, _):
        src_v[pl.ds(g * 16, 16)] = jnp.full((16,), -1, jnp.int32)
        return 0
    lax.fori_loop(0, SRCN // 16, init_body, 0)

    def scan_body(g, _):
        iv = idx_v[pl.ds(g * 16, 16)]
        loc = iv - base
        inr = (loc >= 0) & (loc < RNG)
        j = g * 16 + lane
        _, lastm = plsc.scan_count(loc, mask=inr)
        plsc.store_scatter(src_v, [loc], j, mask=lastm & inr)
        return 0
    lax.fori_loop(0, B // 16, scan_body, 0)

    def comp_body(g, carry):
        k_, pmax_ = carry
        s = src_v[pl.ds(g * 16, 16)]
        m = s >= 0
        ones = jnp.where(m, 1, 0).astype(jnp.int32)
        pos = k_ + plsc.cumsum(ones) - 1
        m = m & (pos < CAP)
        rows = base + g * 16 + lane
        plsc.store_scatter(rowl, [pos >> 6, pos & 63], rows, mask=m)
        plsc.store_scatter(jl, [pos >> 6, pos & 63], s, mask=m)
        pair = jnp.where(m, rows * 16384 + s, -1)
        return k_ + jnp.sum(ones), jnp.maximum(pmax_, jnp.max(pair))
    k, pmax = lax.fori_loop(0, SRCN // 16, comp_body,
                            (jnp.int32(0), jnp.int32(-1)))
    k = jnp.minimum(k, CAP)

    @pl.when(k > 0)
    def _():
        pad_row = pmax >> 14
        pad_j = pmax & 16383

        def pad_body(g, _):
            posv = g * 16 + lane
            m = posv >= k
            plsc.store_scatter(rowl, [posv >> 6, posv & 63],
                               jnp.full((16,), 1, jnp.int32) * pad_row, mask=m)
            plsc.store_scatter(jl, [posv >> 6, posv & 63],
                               jnp.full((16,), 1, jnp.int32) * pad_j, mask=m)
            return 0
        lax.fori_loop(k >> 4, CAP // 16, pad_body, 0)

    npieces = (k + PIECE - 1) >> 6

    pltpu.sync_copy(by_h.at[pl.ds(base, RNG)], cby)
    pltpu.sync_copy(en_h.at[pl.ds(base, RNG)], cen)

    def ye_body(p, _):
        pltpu.sync_copy(y_h.at[jl.at[p]], yv)
        pltpu.sync_copy(eu_h.at[jl.at[p]], ev)
        for g in range(PIECE // 16):
            rows16 = rowl[p, pl.ds(g * 16, 16)] - base
            plsc.store_scatter(cby, [rows16], yv[pl.ds(g * 16, 16)])
            plsc.store_scatter(cen, [rows16], ev[pl.ds(g * 16, 16)])
        return 0
    lax.fori_loop(0, npieces, ye_body, 0)
    pltpu.sync_copy(cby, oby.at[pl.ds(base, RNG)])
    pltpu.sync_copy(cen, oen.at[pl.ds(base, RNG)])

    def wrow(wi):
        pc = jnp.minimum(wi >> 6, NPIECE - 1)
        grp = rowl[pc, pl.ds(((wi >> 4) & 3) * 16, 16)]
        return jnp.sum(jnp.where(lane == (wi & 15), grp, 0))

    def lg_read(c, buf, sem):
        r0 = (base + c * CSL) * C
        return pltpu.make_async_copy(lgf_h.at[pl.ds(r0, CSL * C)], buf, sem)

    def lg_apply(cb, c, carry):
        r0 = base + c * CSL
        r1 = r0 + CSL

        def cond(st):
            wi, _ld = st
            return (wi < k) & (wrow(wi) < r1)

        def body(st):
            wi, ld = st
            pc = wi >> 6

            @pl.when(pc != ld)
            def _():
                pltpu.sync_copy(xlgp_h.at[jl.at[pc]], lgpiece)
            off = (wrow(wi) - r0) * C
            for t in range(7):
                o = min(t * 16, C - 16)
                src = lgpiece[wi & 63, pl.ds(o, 16)]
                plsc.store_scatter(cb, [off + o + lane], src)
            return wi + 1, pc
        return lax.while_loop(cond, body, carry)

    lg_read(0, clg0, sem0).start()

    def lg_pair(cp, carry):
        c0 = cp * 2
        c1 = c0 + 1
        lg_read(c0, clg0, sem0).wait()

        @pl.when(c1 < NCHL)
        def _():
            lg_read(c1, clg1, sem1).start()
        carry = lg_apply(clg0, c0, carry)
        pltpu.sync_copy(clg0, olgf.at[pl.ds((base + c0 * CSL) * C, CSL * C)])

        def odd(carry):
            lg_read(c1, clg1, sem1).wait()

            @pl.when(c1 + 1 < NCHL)
            def _():
                lg_read(c1 + 1, clg0, sem0).start()
            carry = lg_apply(clg1, c1, carry)
            pltpu.sync_copy(clg1,
                            olgf.at[pl.ds((base + c1 * CSL) * C, CSL * C)])
            return carry
        return lax.cond(c1 < NCHL, odd, lambda carry: carry, carry)

    lax.fori_loop(0, (NCHL + 1) // 2, lg_pair, (jnp.int32(0), jnp.int32(-1)))

    def bx_read(c, buf, sem):
        return pltpu.make_async_copy(bx_h.at[pl.ds(base + c * CSX, CSX)],
                                     buf, sem)

    def bx_apply(cb, c, carry):
        r0 = base + c * CSX
        r1 = r0 + CSX

        def cond(st):
            wi, _ld = st
            return (wi < k) & (wrow(wi) < r1)

        def body(st):
            wi, ld = st
            pc = wi >> 6

            @pl.when(pc != ld)
            def _():
                pltpu.sync_copy(x_h.at[jl.at[pc]], xpiece)
            lr = wrow(wi) - r0
            for t in range(D // 16):
                cb[lr, pl.ds(t * 16, 16)] = xpiece[wi & 63, pl.ds(t * 16, 16)]
            return wi + 1, pc
        return lax.while_loop(cond, body, carry)

    bx_read(0, cbx0, sem0).start()

    def bx_pair(cp, carry):
        c0 = cp * 2
        c1 = c0 + 1
        bx_read(c0, cbx0, sem0).wait()
        bx_read(c1, cbx1, sem1).start()
        carry = bx_apply(cbx0, c0, carry)
        pltpu.sync_copy(cbx0, obx.at[pl.ds(base + c0 * CSX, CSX)])
        bx_read(c1, cbx1, sem1).wait()

        @pl.when(c1 + 1 < NCHX)
        def _():
            bx_read(c1 + 1, cbx0, sem0).start()
        carry = bx_apply(cbx1, c1, carry)
        pltpu.sync_copy(cbx1, obx.at[pl.ds(base + c1 * CSX, CSX)])
        return carry

    lax.fori_loop(0, NCHX // 2, bx_pair, (jnp.int32(0), jnp.int32(-1)))


_sc_all = pl.kernel(
    _sc_body,
    out_type=(jax.ShapeDtypeStruct((M, D), jnp.float32),
              jax.ShapeDtypeStruct((M,), jnp.int32),
              jax.ShapeDtypeStruct((M,), jnp.float32),
              jax.ShapeDtypeStruct((M * C,), jnp.float32)),
    mesh=plsc.VectorSubcoreMesh(core_axis_name="c", subcore_axis_name="s"),
    compiler_params=pltpu.CompilerParams(needs_layout_passes=False,
                                         use_tc_tiling_on_sc=False),
    scratch_types=[
        pltpu.VMEM((B,), jnp.int32),
        pltpu.VMEM((SRCN,), jnp.int32),
        pltpu.VMEM((NPIECE, PIECE), jnp.int32),
        pltpu.VMEM((NPIECE, PIECE), jnp.int32),
        pltpu.VMEM((PIECE,), jnp.int32),
        pltpu.VMEM((PIECE,), jnp.float32),
        pltpu.VMEM((CSX, D), jnp.float32),
        pltpu.VMEM((CSX, D), jnp.float32),
        pltpu.VMEM((CSL * C,), jnp.float32),
        pltpu.VMEM((CSL * C,), jnp.float32),
        pltpu.VMEM((PIECE, D), jnp.float32),
        pltpu.VMEM((PIECE, CP), jnp.float32),
        pltpu.VMEM((RNG,), jnp.int32),
        pltpu.VMEM((RNG,), jnp.float32),
        pltpu.SemaphoreType.DMA,
        pltpu.SemaphoreType.DMA,
    ],
)


@jax.jit
def kernel(bx, by_buf, ents_buf, logits_buf, x, y, ents, logits, idx):
    xlgp = _tc_padx(logits)
    obx, oby, oen, olgf = _sc_all(bx, by_buf, ents_buf,
                                  logits_buf.reshape(M * C),
                                  x, y, ents, xlgp, idx)
    return obx, oby, oen, olgf.reshape(M, C)


# speedup vs baseline: 1.1195x; 1.1172x over previous
import jax
import jax.numpy as jnp
from jax import lax
from jax.experimental import pallas as pl
from jax.experimental.pallas import tpu as pltpu
from jax.experimental.pallas import tpu_sc as plsc

M = 100000
D = 256
B = 16384
C = 100
CP = 128

NW = 32
RNG = 3128
CAP = 1024
PIECE = 128
NPIECE = CAP // PIECE
SRCN = 3136


def _copy_body(bx_r, lg_r, by_r, en_r,
               obx_r, olg_r, oby_r, oen_r):
    obx_r[...] = bx_r[...]
    olg_r[:, :C] = lg_r[...]
    oby_r[...] = by_r[...]
    oen_r[...] = en_r[...]


def _tc_copy(bx, logits_buf, by2, en2):
    grid = 100
    bm = M // grid
    return pl.pallas_call(
        _copy_body,
        grid=(grid,),
        in_specs=[
            pl.BlockSpec((bm, D), lambda i: (i, 0)),
            pl.BlockSpec((bm, C), lambda i: (i, 0)),
            pl.BlockSpec((8, 125), lambda i: (i, 0)),
            pl.BlockSpec((8, 125), lambda i: (i, 0)),
        ],
        out_specs=[
            pl.BlockSpec((bm, D), lambda i: (i, 0)),
            pl.BlockSpec((bm, CP), lambda i: (i, 0)),
            pl.BlockSpec((8, 125), lambda i: (i, 0)),
            pl.BlockSpec((8, 125), lambda i: (i, 0)),
        ],
        out_shape=[
            jax.ShapeDtypeStruct((M, D), jnp.float32),
            jax.ShapeDtypeStruct((M, CP), jnp.float32),
            jax.ShapeDtypeStruct((800, 125), jnp.int32),
            jax.ShapeDtypeStruct((800, 125), jnp.float32),
        ],
    )(bx, logits_buf, by2, en2)


def _padx_body(xlg_r, oxlg_r):
    oxlg_r[:, :C] = xlg_r[...]


def _tc_padx(xlg):
    grid = 128
    bb = B // grid
    return pl.pallas_call(
        _padx_body,
        grid=(grid,),
        in_specs=[pl.BlockSpec((bb, C), lambda i: (i, 0))],
        out_specs=pl.BlockSpec((bb, CP), lambda i: (i, 0)),
        out_shape=jax.ShapeDtypeStruct((B, CP), jnp.float32),
    )(xlg)


def _unpad_body(p_r, o_r):
    o_r[...] = p_r[:, :C]


def _tc_unpad(p):
    grid = 100
    bm = M // grid
    return pl.pallas_call(
        _unpad_body,
        grid=(grid,),
        in_specs=[pl.BlockSpec((bm, CP), lambda i: (i, 0))],
        out_specs=pl.BlockSpec((bm, C), lambda i: (i, 0)),
        out_shape=jax.ShapeDtypeStruct((M, C), jnp.float32),
    )(p)


def _sc_body(x_hbm, y_hbm, e_hbm, lg_hbm, idx_hbm,
             rbx, rby, ren, rlg,
             idx_v, src_v, rowl, jl, yv, ev, xbuf, lbuf, semg, sems):
    w = lax.axis_index("s") * 2 + lax.axis_index("c")
    base = jnp.where(w == NW - 1, M - RNG, w * RNG).astype(jnp.int32)
    lane = lax.iota(jnp.int32, 16)

    pltpu.sync_copy(idx_hbm, idx_v)

    def init_body(g, _):
        src_v[pl.ds(g * 16, 16)] = jnp.full((16,), -1, jnp.int32)
        return 0
    lax.fori_loop(0, SRCN // 16, init_body, 0)

    def scan_body(g, _):
        iv = idx_v[pl.ds(g * 16, 16)]
        loc = iv - base
        inr = (loc >= 0) & (loc < RNG)
        j = g * 16 + lane
        _, lastm = plsc.scan_count(loc, mask=inr)
        plsc.store_scatter(src_v, [loc], j, mask=lastm & inr)
        return 0
    lax.fori_loop(0, B // 16, scan_body, 0)

    def comp_body(g, carry):
        k, pmax = carry
        s = src_v[pl.ds(g * 16, 16)]
        m = s >= 0
        ones = jnp.where(m, 1, 0).astype(jnp.int32)
        pos = k + plsc.cumsum(ones) - 1
        m = m & (pos < CAP)
        rows = base + g * 16 + lane
        plsc.store_scatter(rowl, [pos >> 7, pos & 127], rows, mask=m)
        plsc.store_scatter(jl, [pos >> 7, pos & 127], s, mask=m)
        pair = jnp.where(m, rows * 16384 + s, -1)
        return k + jnp.sum(ones), jnp.maximum(pmax, jnp.max(pair))
    k, pmax = lax.fori_loop(0, SRCN // 16, comp_body,
                            (jnp.int32(0), jnp.int32(-1)))

    @pl.when(k > 0)
    def _():
        pad_row = pmax >> 14
        pad_j = pmax & 16383

        def pad_body(g, _):
            posv = g * 16 + lane
            m = posv >= k
            plsc.store_scatter(rowl, [posv >> 7, posv & 127],
                               jnp.full((16,), 1, jnp.int32) * pad_row, mask=m)
            plsc.store_scatter(jl, [posv >> 7, posv & 127],
                               jnp.full((16,), 1, jnp.int32) * pad_j, mask=m)
            return 0
        lax.fori_loop(k >> 4, CAP // 16, pad_body, 0)

        def upd_body(p, _):
            pltpu.make_async_copy(x_hbm.at[jl.at[p]], xbuf, semg).start()
            pltpu.make_async_copy(lg_hbm.at[jl.at[p]], lbuf, semg).start()
            pltpu.make_async_copy(y_hbm.at[jl.at[p]], yv, semg).start()
            pltpu.make_async_copy(e_hbm.at[jl.at[p]], ev, semg).start()
            pltpu.make_async_copy(x_hbm.at[jl.at[p]], xbuf, semg).wait()
            pltpu.make_async_copy(lg_hbm.at[jl.at[p]], lbuf, semg).wait()
            pltpu.make_async_copy(y_hbm.at[jl.at[p]], yv, semg).wait()
            pltpu.make_async_copy(e_hbm.at[jl.at[p]], ev, semg).wait()
            pltpu.make_async_copy(xbuf, rbx.at[rowl.at[p]], sems).start()
            pltpu.make_async_copy(lbuf, rlg.at[rowl.at[p]], sems).start()
            pltpu.make_async_copy(yv, rby.at[rowl.at[p]], sems).start()
            pltpu.make_async_copy(ev, ren.at[rowl.at[p]], sems).start()
            pltpu.make_async_copy(xbuf, rbx.at[rowl.at[p]], sems).wait()
            pltpu.make_async_copy(lbuf, rlg.at[rowl.at[p]], sems).wait()
            pltpu.make_async_copy(yv, rby.at[rowl.at[p]], sems).wait()
            pltpu.make_async_copy(ev, ren.at[rowl.at[p]], sems).wait()
            return 0
        lax.fori_loop(0, (k + PIECE - 1) >> 7, upd_body, 0)


_sc_update = pl.kernel(
    _sc_body,
    out_type=(),
    mesh=plsc.VectorSubcoreMesh(core_axis_name="c", subcore_axis_name="s"),
    compiler_params=pltpu.CompilerParams(needs_layout_passes=False,
                                         use_tc_tiling_on_sc=False),
    scratch_types=[
        pltpu.VMEM((B,), jnp.int32),
        pltpu.VMEM((SRCN,), jnp.int32),
        pltpu.VMEM((NPIECE, PIECE), jnp.int32),
        pltpu.VMEM((NPIECE, PIECE), jnp.int32),
        pltpu.VMEM((PIECE,), jnp.int32),
        pltpu.VMEM((PIECE,), jnp.float32),
        pltpu.VMEM((PIECE, D), jnp.float32),
        pltpu.VMEM((PIECE, CP), jnp.float32),
        pltpu.SemaphoreType.DMA,
        pltpu.SemaphoreType.DMA,
    ],
)


@jax.jit
def kernel(bx, by_buf, ents_buf, logits_buf, x, y, ents, logits, idx):
    xlgp = _tc_padx(logits)
    rbx = jax.new_ref(bx)
    rby = jax.new_ref(by_buf)
    ren = jax.new_ref(ents_buf)
    rlg = jax.new_ref(jnp.pad(logits_buf, ((0, 0), (0, CP - C))))
    _sc_update(x, y, ents, xlgp, idx, rbx, rby, ren, rlg)
    return rbx[...], rby[...], ren[...], _tc_unpad(rlg[...])


# speedup vs baseline: 1.2953x; 1.1571x over previous
import jax
import jax.numpy as jnp
from jax import lax
from jax.experimental import pallas as pl
from jax.experimental.pallas import tpu as pltpu
from jax.experimental.pallas import tpu_sc as plsc

M = 100000
D = 256
B = 16384
C = 100
CP = 128

NW = 32
RNG = 3128
CAP = 1024
PIECE = 128
NPIECE = CAP // PIECE
SRCN = 3136


def _copy_body(bx_r, lg_r, by_r, en_r,
               obx_r, olg_r, oby_r, oen_r):
    obx_r[...] = bx_r[...]
    olg_r[:, :C] = lg_r[...]
    oby_r[...] = by_r[...]
    oen_r[...] = en_r[...]


def _tc_copy(bx, logits_buf, by2, en2):
    grid = 100
    bm = M // grid
    return pl.pallas_call(
        _copy_body,
        grid=(grid,),
        in_specs=[
            pl.BlockSpec((bm, D), lambda i: (i, 0)),
            pl.BlockSpec((bm, C), lambda i: (i, 0)),
            pl.BlockSpec((8, 125), lambda i: (i, 0)),
            pl.BlockSpec((8, 125), lambda i: (i, 0)),
        ],
        out_specs=[
            pl.BlockSpec((bm, D), lambda i: (i, 0)),
            pl.BlockSpec((bm, CP), lambda i: (i, 0)),
            pl.BlockSpec((8, 125), lambda i: (i, 0)),
            pl.BlockSpec((8, 125), lambda i: (i, 0)),
        ],
        out_shape=[
            jax.ShapeDtypeStruct((M, D), jnp.float32),
            jax.ShapeDtypeStruct((M, CP), jnp.float32),
            jax.ShapeDtypeStruct((800, 125), jnp.int32),
            jax.ShapeDtypeStruct((800, 125), jnp.float32),
        ],
    )(bx, logits_buf, by2, en2)


def _padx_body(xlg_r, oxlg_r):
    oxlg_r[:, :C] = xlg_r[...]


def _tc_padx(xlg):
    grid = 128
    bb = B // grid
    return pl.pallas_call(
        _padx_body,
        grid=(grid,),
        in_specs=[pl.BlockSpec((bb, C), lambda i: (i, 0))],
        out_specs=pl.BlockSpec((bb, CP), lambda i: (i, 0)),
        out_shape=jax.ShapeDtypeStruct((B, CP), jnp.float32),
    )(xlg)


def _unpad_body(p_r, o_r):
    o_r[...] = p_r[:, :C]


def _tc_unpad(p):
    grid = 100
    bm = M // grid
    return pl.pallas_call(
        _unpad_body,
        grid=(grid,),
        in_specs=[pl.BlockSpec((bm, CP), lambda i: (i, 0))],
        out_specs=pl.BlockSpec((bm, C), lambda i: (i, 0)),
        out_shape=jax.ShapeDtypeStruct((M, C), jnp.float32),
    )(p)


def _sc_body(x_hbm, y_hbm, e_hbm, lg_hbm, idx_hbm,
             rbx, rby, ren, rlg,
             idx_v, src_v, rowl, jl, yv, ev, xbuf, lbuf, semg, sems):
    w = lax.axis_index("s") * 2 + lax.axis_index("c")
    base = jnp.where(w == NW - 1, M - RNG, w * RNG).astype(jnp.int32)
    lane = lax.iota(jnp.int32, 16)

    pltpu.sync_copy(idx_hbm, idx_v)

    def init_body(g, _):
        src_v[pl.ds(g * 16, 16)] = jnp.full((16,), -1, jnp.int32)
        return 0
    lax.fori_loop(0, SRCN // 16, init_body, 0)

    def scan_body(g, _):
        iv = idx_v[pl.ds(g * 16, 16)]
        loc = iv - base
        inr = (loc >= 0) & (loc < RNG)
        j = g * 16 + lane
        _, lastm = plsc.scan_count(loc, mask=inr)
        plsc.store_scatter(src_v, [loc], j, mask=lastm & inr)
        return 0
    lax.fori_loop(0, B // 16, scan_body, 0)

    def comp_body(g, carry):
        k, pmax = carry
        s = src_v[pl.ds(g * 16, 16)]
        m = s >= 0
        ones = jnp.where(m, 1, 0).astype(jnp.int32)
        pos = k + plsc.cumsum(ones) - 1
        m = m & (pos < CAP)
        rows = base + g * 16 + lane
        plsc.store_scatter(rowl, [pos >> 7, pos & 127], rows, mask=m)
        plsc.store_scatter(jl, [pos >> 7, pos & 127], s, mask=m)
        pair = jnp.where(m, rows * 16384 + s, -1)
        return k + jnp.sum(ones), jnp.maximum(pmax, jnp.max(pair))
    k, pmax = lax.fori_loop(0, SRCN // 16, comp_body,
                            (jnp.int32(0), jnp.int32(-1)))

    @pl.when(k > 0)
    def _():
        pad_row = pmax >> 14
        pad_j = pmax & 16383

        def pad_body(g, _):
            posv = g * 16 + lane
            m = posv >= k
            plsc.store_scatter(rowl, [posv >> 7, posv & 127],
                               jnp.full((16,), 1, jnp.int32) * pad_row, mask=m)
            plsc.store_scatter(jl, [posv >> 7, posv & 127],
                               jnp.full((16,), 1, jnp.int32) * pad_j, mask=m)
            return 0
        lax.fori_loop(k >> 4, CAP // 16, pad_body, 0)

        def upd_body(p, _):
            pltpu.make_async_copy(x_hbm.at[jl.at[p]], xbuf, semg).start()
            pltpu.make_async_copy(lg_hbm.at[jl.at[p]], lbuf, semg).start()
            pltpu.make_async_copy(y_hbm.at[jl.at[p]], yv, semg).start()
            pltpu.make_async_copy(e_hbm.at[jl.at[p]], ev, semg).start()
            pltpu.make_async_copy(x_hbm.at[jl.at[p]], xbuf, semg).wait()
            pltpu.make_async_copy(lg_hbm.at[jl.at[p]], lbuf, semg).wait()
            pltpu.make_async_copy(y_hbm.at[jl.at[p]], yv, semg).wait()
            pltpu.make_async_copy(e_hbm.at[jl.at[p]], ev, semg).wait()
            pltpu.make_async_copy(xbuf, rbx.at[rowl.at[p]], sems).start()
            pltpu.make_async_copy(lbuf, rlg.at[rowl.at[p]], sems).start()
            pltpu.make_async_copy(yv, rby.at[rowl.at[p]], sems).start()
            pltpu.make_async_copy(ev, ren.at[rowl.at[p]], sems).start()
            pltpu.make_async_copy(xbuf, rbx.at[rowl.at[p]], sems).wait()
            pltpu.make_async_copy(lbuf, rlg.at[rowl.at[p]], sems).wait()
            pltpu.make_async_copy(yv, rby.at[rowl.at[p]], sems).wait()
            pltpu.make_async_copy(ev, ren.at[rowl.at[p]], sems).wait()
            return 0
        lax.fori_loop(0, (k + PIECE - 1) >> 7, upd_body, 0)


_sc_update = pl.kernel(
    _sc_body,
    out_type=(),
    mesh=plsc.VectorSubcoreMesh(core_axis_name="c", subcore_axis_name="s"),
    compiler_params=pltpu.CompilerParams(needs_layout_passes=False,
                                         use_tc_tiling_on_sc=False),
    scratch_types=[
        pltpu.VMEM((B,), jnp.int32),
        pltpu.VMEM((SRCN,), jnp.int32),
        pltpu.VMEM((NPIECE, PIECE), jnp.int32),
        pltpu.VMEM((NPIECE, PIECE), jnp.int32),
        pltpu.VMEM((PIECE,), jnp.int32),
        pltpu.VMEM((PIECE,), jnp.float32),
        pltpu.VMEM((PIECE, D), jnp.float32),
        pltpu.VMEM((PIECE, CP), jnp.float32),
        pltpu.SemaphoreType.DMA,
        pltpu.SemaphoreType.DMA,
    ],
)


@jax.jit
def kernel(bx, by_buf, ents_buf, logits_buf, x, y, ents, logits, idx):
    xlgp = jnp.pad(logits, ((0, 0), (0, CP - C)))
    rbx = jax.new_ref(bx)
    rby = jax.new_ref(by_buf)
    ren = jax.new_ref(ents_buf)
    rlg = jax.new_ref(jnp.pad(logits_buf, ((0, 0), (0, CP - C))))
    _sc_update(x, y, ents, xlgp, idx, rbx, rby, ren, rlg)
    return rbx[...], rby[...], ren[...], rlg[...][:, :C]
